# baseline (speedup 1.0000x reference)
import functools

import jax
import jax.numpy as jnp
from jax.experimental import pallas as pl
from jax.experimental.pallas import tpu as pltpu
from jax.experimental.pallas import tpu_sc as plsc

B, S, D = 1, 2048, 768
E, TOPK, FFN = 8, 2, 3072
T = B * S
BLK = 256
TILES = (T * TOPK + E * BLK) // BLK
PMAX = TILES * BLK
CAPACITY = max(1, int(T * TOPK * 1.25) // E)
F32 = jnp.float32


def _router_body(x_ref, g_ref, b_ref, wg_ref,
                 xn_ref, logits_ref, dw_ref, ei_ref, p0_ref, p1_ref,
                 w0_ref, w1_ref, eot_ref,
                 lbl_ref, rel_ref, nd_ref, usage_ref):
    x = x_ref[...]
    mean = jnp.mean(x, axis=1, keepdims=True)
    cx = x - mean
    var = jnp.mean(cx * cx, axis=1, keepdims=True)
    xn = cx * jax.lax.rsqrt(var + 1e-5) * g_ref[...] + b_ref[...]
    xn_ref[...] = xn

    logits = jnp.dot(xn, wg_ref[...], preferred_element_type=F32)
    logits_ref[...] = logits
    m = jnp.max(logits, axis=1, keepdims=True)
    ex = jnp.exp(logits - m)
    den = jnp.sum(ex, axis=1, keepdims=True)
    probs = ex / den

    idx = jax.lax.broadcasted_iota(jnp.int32, (T, E), 1)
    p0 = jnp.max(probs, axis=1, keepdims=True)
    is0 = probs == p0
    e0 = jnp.min(jnp.where(is0, idx, E), axis=1, keepdims=True)
    pm = jnp.where(idx == e0, -1.0, probs)
    p1 = jnp.max(pm, axis=1, keepdims=True)
    is1 = pm == p1
    e1 = jnp.min(jnp.where(is1, idx, E), axis=1, keepdims=True)
    dw_ref[...] = jnp.concatenate([p0, p1], axis=1)
    ei_ref[...] = jnp.concatenate([e0, e1], axis=1)

    logp = logits - m - jnp.log(den)
    ent = -jnp.sum(probs * logp, axis=1, keepdims=True)
    entropy = jnp.mean(ent)
    rel_ref[...] = jnp.maximum(jnp.log(jnp.float32(E)) - entropy,
                               0.0).reshape(1, 1)
    avg_probs = jnp.mean(probs, axis=0, keepdims=True)

    oh0 = (idx == e0).astype(F32)
    oh1 = (idx == e1).astype(F32)
    r = jax.lax.broadcasted_iota(jnp.int32, (T, T), 0)
    c = jax.lax.broadcasted_iota(jnp.int32, (T, T), 1)
    tri = (r >= c).astype(F32)
    cums = jnp.dot(tri, jnp.concatenate([oh0, oh1], axis=1),
                   preferred_element_type=F32)
    cum0 = cums[:, :E]
    cum1 = cums[:, E:]
    counts0 = cum0[T - 1:T, :]
    counts1 = cum1[T - 1:T, :]
    counts = counts0 + counts1

    nd_ref[...] = jnp.sum((counts > CAPACITY).astype(F32)
                          ).astype(jnp.int32).reshape(1, 1)
    usage = counts / T
    usage_ref[...] = usage
    lbl_ref[...] = (jnp.sum(usage * avg_probs) * E).reshape(1, 1)

    aligned = jnp.ceil(counts / BLK) * BLK
    r8 = jax.lax.broadcasted_iota(jnp.int32, (E, E), 0)
    c8 = jax.lax.broadcasted_iota(jnp.int32, (E, E), 1)
    up = (r8 < c8).astype(F32)
    offs = jnp.dot(aligned, up, preferred_element_type=F32)
    ends = offs + aligned

    pos0 = jnp.sum(oh0 * (cum0 + offs), axis=1, keepdims=True) - 1.0
    pos1 = jnp.sum(oh1 * (cum1 + counts0 + offs), axis=1, keepdims=True) - 1.0
    p0_ref[...] = pos0.astype(jnp.int32)
    p1_ref[...] = pos1.astype(jnp.int32)
    w0_ref[...] = p0
    w1_ref[...] = p1

    ts = (jax.lax.broadcasted_iota(jnp.int32, (TILES, E), 0).astype(F32) * BLK
          >= jax.lax.broadcast_in_dim(ends, (TILES, E), (0, 1))).astype(F32)
    eot_ref[...] = jnp.sum(ts, axis=1, keepdims=True).astype(jnp.int32)


def _router(xf, gamma, beta, Wg):
    outs = (
        jax.ShapeDtypeStruct((T, D), F32),
        jax.ShapeDtypeStruct((T, E), F32),
        jax.ShapeDtypeStruct((T, TOPK), F32),
        jax.ShapeDtypeStruct((T, TOPK), jnp.int32),
        jax.ShapeDtypeStruct((T, 1), jnp.int32),
        jax.ShapeDtypeStruct((T, 1), jnp.int32),
        jax.ShapeDtypeStruct((T, 1), F32),
        jax.ShapeDtypeStruct((T, 1), F32),
        jax.ShapeDtypeStruct((TILES, 1), jnp.int32),
        jax.ShapeDtypeStruct((1, 1), F32),
        jax.ShapeDtypeStruct((1, 1), F32),
        jax.ShapeDtypeStruct((1, 1), jnp.int32),
        jax.ShapeDtypeStruct((1, E), F32),
    )
    return pl.pallas_call(_router_body, out_shape=outs)(
        xf, gamma.reshape(1, D), beta.reshape(1, D), Wg)


_NC, _NS = 2, 16
_NW = _NC * _NS
_TPW = T // _NW


def _sc_mesh():
    return plsc.VectorSubcoreMesh(core_axis_name="c", subcore_axis_name="s")


def _sc_dispatch_body(xn_hbm, p0_hbm, p1_hbm, w0_hbm, w1_hbm,
                      rows_hbm, ws_hbm, idx0_v, idx1_v, rows_v,
                      wv0_v, wv1_v, sem):
    wid = jax.lax.axis_index("s") * _NC + jax.lax.axis_index("c")
    base = wid * _TPW
    pltpu.sync_copy(p0_hbm.at[pl.ds(base, _TPW)], idx0_v)
    pltpu.sync_copy(p1_hbm.at[pl.ds(base, _TPW)], idx1_v)
    pltpu.sync_copy(w0_hbm.at[pl.ds(base, _TPW)], wv0_v)
    pltpu.sync_copy(w1_hbm.at[pl.ds(base, _TPW)], wv1_v)
    pltpu.sync_copy(xn_hbm.at[pl.ds(base, _TPW)], rows_v)
    c0 = pltpu.async_copy(rows_v, rows_hbm.at[idx0_v], sem)
    c1 = pltpu.async_copy(rows_v, rows_hbm.at[idx1_v], sem)
    c2 = pltpu.async_copy(wv0_v, ws_hbm.at[idx0_v], sem)
    c3 = pltpu.async_copy(wv1_v, ws_hbm.at[idx1_v], sem)
    c0.wait()
    c1.wait()
    c2.wait()
    c3.wait()


def _dispatch(xn, p0, p1, w0, w1):
    return pl.kernel(
        _sc_dispatch_body,
        out_type=(jax.ShapeDtypeStruct((PMAX, D), F32),
                  jax.ShapeDtypeStruct((PMAX,), F32)),
        mesh=_sc_mesh(),
        scratch_types=[
            pltpu.VMEM((_TPW,), jnp.int32),
            pltpu.VMEM((_TPW,), jnp.int32),
            pltpu.VMEM((_TPW, D), F32),
            pltpu.VMEM((_TPW,), F32),
            pltpu.VMEM((_TPW,), F32),
            pltpu.SemaphoreType.DMA,
        ],
    )(xn, p0, p1, w0, w1)


_SQRT_HALF = 0.7071067811865476


def _ffn_body(eot_ref, rows_ref, ws_ref, w1_ref, b1_ref, w2_ref, b2_ref,
              h_ref):
    i = pl.program_id(0)

    @pl.when(eot_ref[i] < E)
    def _():
        a = (jnp.dot(rows_ref[...], w1_ref[0], preferred_element_type=F32)
             + b1_ref[0])
        g = a * 0.5 * (1.0 + jax.lax.erf(a * _SQRT_HALF))
        h = jnp.dot(g, w2_ref[0], preferred_element_type=F32) + b2_ref[0]
        h_ref[...] = h * ws_ref[...]


def _ffn(eot, rows, ws, w1, b1, w2, b2):
    grid_spec = pltpu.PrefetchScalarGridSpec(
        num_scalar_prefetch=1,
        grid=(TILES,),
        in_specs=[
            pl.BlockSpec((BLK, D), lambda i, eot: (i, 0)),
            pl.BlockSpec((BLK, 1), lambda i, eot: (i, 0)),
            pl.BlockSpec((1, D, FFN), lambda i, eot: (jnp.minimum(eot[i], E - 1), 0, 0)),
            pl.BlockSpec((1, 1, FFN), lambda i, eot: (jnp.minimum(eot[i], E - 1), 0, 0)),
            pl.BlockSpec((1, FFN, D), lambda i, eot: (jnp.minimum(eot[i], E - 1), 0, 0)),
            pl.BlockSpec((1, 1, D), lambda i, eot: (jnp.minimum(eot[i], E - 1), 0, 0)),
        ],
        out_specs=pl.BlockSpec((BLK, D), lambda i, eot: (i, 0)),
    )
    return pl.pallas_call(
        _ffn_body,
        grid_spec=grid_spec,
        out_shape=jax.ShapeDtypeStruct((PMAX, D), F32),
    )(eot, rows, ws.reshape(PMAX, 1), w1, b1.reshape(E, 1, FFN), w2,
      b2.reshape(E, 1, D))


_TPC = 32


def _sc_combine_body(h_hbm, x_hbm, p0_hbm, p1_hbm, out_hbm,
                     idx0_v, idx1_v, x_v, r0_v, r1_v, sem):
    wid = jax.lax.axis_index("s") * _NC + jax.lax.axis_index("c")
    base = wid * _TPW
    for c in range(_TPW // _TPC):
        cb = base + c * _TPC
        pltpu.sync_copy(p0_hbm.at[pl.ds(cb, _TPC)], idx0_v)
        pltpu.sync_copy(p1_hbm.at[pl.ds(cb, _TPC)], idx1_v)
        g0 = pltpu.async_copy(h_hbm.at[idx0_v], r0_v, sem)
        g1 = pltpu.async_copy(h_hbm.at[idx1_v], r1_v, sem)
        pltpu.sync_copy(x_hbm.at[pl.ds(cb, _TPC)], x_v)
        g0.wait()
        g1.wait()

        def body(i, carry):
            for j in range(D // 16):
                sl = pl.ds(j * 16, 16)
                x_v[i, sl] = x_v[i, sl] + r0_v[i, sl] + r1_v[i, sl]
            return carry

        jax.lax.fori_loop(0, _TPC, body, 0)
        pltpu.sync_copy(x_v, out_hbm.at[pl.ds(cb, _TPC)])


def _combine(h, xf, p0, p1):
    return pl.kernel(
        _sc_combine_body,
        out_type=jax.ShapeDtypeStruct((T, D), F32),
        mesh=_sc_mesh(),
        scratch_types=[
            pltpu.VMEM((_TPC,), jnp.int32),
            pltpu.VMEM((_TPC,), jnp.int32),
            pltpu.VMEM((_TPC, D), F32),
            pltpu.VMEM((_TPC, D), F32),
            pltpu.VMEM((_TPC, D), F32),
            pltpu.SemaphoreType.DMA,
        ],
    )(h, xf, p0, p1)


def kernel(x, gamma, beta, Wg, w1, b1, w2, b2):
    xf = x.reshape(T, D)
    (xn, logits, dw, ei, p0, p1, dw0, dw1, eot,
     lbl, rel, nd, usage) = _router(xf, gamma, beta, Wg)
    p0f = p0.reshape(T)
    p1f = p1.reshape(T)
    rows, ws = _dispatch(xn, p0f, p1f, dw0.reshape(T), dw1.reshape(T))
    h = _ffn(eot.reshape(TILES), rows, ws, w1, b1, w2, b2)
    out = _combine(h, xf, p0f, p1f).reshape(B, S, D)
    return (out, dw, ei, logits, lbl[0, 0], rel[0, 0], nd[0, 0], usage[0])


# speedup vs baseline: 1.0509x; 1.0509x over previous
import functools

import jax
import jax.numpy as jnp
from jax.experimental import pallas as pl
from jax.experimental.pallas import tpu as pltpu
from jax.experimental.pallas import tpu_sc as plsc

B, S, D = 1, 2048, 768
E, TOPK, FFN = 8, 2, 3072
T = B * S
BLK = 256
TILES = (T * TOPK + E * BLK) // BLK
PMAX = TILES * BLK
CAPACITY = max(1, int(T * TOPK * 1.25) // E)
F32 = jnp.float32


def _router_body(x_ref, g_ref, b_ref, wg_ref,
                 xn_ref, logits_ref, dw_ref, ei_ref, p0_ref, p1_ref,
                 w0_ref, w1_ref, eot_ref,
                 lbl_ref, rel_ref, nd_ref, usage_ref):
    x = x_ref[...]
    mean = jnp.mean(x, axis=1, keepdims=True)
    cx = x - mean
    var = jnp.mean(cx * cx, axis=1, keepdims=True)
    xn = cx * jax.lax.rsqrt(var + 1e-5) * g_ref[...] + b_ref[...]
    xn_ref[...] = xn

    logits = jnp.dot(xn, wg_ref[...], preferred_element_type=F32)
    logits_ref[...] = logits
    m = jnp.max(logits, axis=1, keepdims=True)
    ex = jnp.exp(logits - m)
    den = jnp.sum(ex, axis=1, keepdims=True)
    probs = ex / den

    idx = jax.lax.broadcasted_iota(jnp.int32, (T, E), 1)
    p0 = jnp.max(probs, axis=1, keepdims=True)
    is0 = probs == p0
    e0 = jnp.min(jnp.where(is0, idx, E), axis=1, keepdims=True)
    pm = jnp.where(idx == e0, -1.0, probs)
    p1 = jnp.max(pm, axis=1, keepdims=True)
    is1 = pm == p1
    e1 = jnp.min(jnp.where(is1, idx, E), axis=1, keepdims=True)
    dw_ref[...] = jnp.concatenate([p0, p1], axis=1)
    ei_ref[...] = jnp.concatenate([e0, e1], axis=1)

    logp = logits - m - jnp.log(den)
    ent = -jnp.sum(probs * logp, axis=1, keepdims=True)
    entropy = jnp.mean(ent)
    rel_ref[...] = jnp.maximum(jnp.log(jnp.float32(E)) - entropy,
                               0.0).reshape(1, 1)
    avg_probs = jnp.mean(probs, axis=0, keepdims=True)

    oh0 = (idx == e0).astype(F32)
    oh1 = (idx == e1).astype(F32)
    r = jax.lax.broadcasted_iota(jnp.int32, (T, T), 0)
    c = jax.lax.broadcasted_iota(jnp.int32, (T, T), 1)
    tri = (r >= c).astype(F32)
    cums = jnp.dot(tri, jnp.concatenate([oh0, oh1], axis=1),
                   preferred_element_type=F32)
    cum0 = cums[:, :E]
    cum1 = cums[:, E:]
    counts0 = cum0[T - 1:T, :]
    counts1 = cum1[T - 1:T, :]
    counts = counts0 + counts1

    nd_ref[...] = jnp.sum((counts > CAPACITY).astype(F32)
                          ).astype(jnp.int32).reshape(1, 1)
    usage = counts / T
    usage_ref[...] = usage
    lbl_ref[...] = (jnp.sum(usage * avg_probs) * E).reshape(1, 1)

    aligned = jnp.ceil(counts / BLK) * BLK
    r8 = jax.lax.broadcasted_iota(jnp.int32, (E, E), 0)
    c8 = jax.lax.broadcasted_iota(jnp.int32, (E, E), 1)
    up = (r8 < c8).astype(F32)
    offs = jnp.dot(aligned, up, preferred_element_type=F32)
    ends = offs + aligned

    pos0 = jnp.sum(oh0 * (cum0 + offs), axis=1, keepdims=True) - 1.0
    pos1 = jnp.sum(oh1 * (cum1 + counts0 + offs), axis=1, keepdims=True) - 1.0
    p0_ref[...] = pos0.astype(jnp.int32)
    p1_ref[...] = pos1.astype(jnp.int32)
    w0_ref[...] = p0
    w1_ref[...] = p1

    ts = (jax.lax.broadcasted_iota(jnp.int32, (TILES, E), 0).astype(F32) * BLK
          >= jax.lax.broadcast_in_dim(ends, (TILES, E), (0, 1))).astype(F32)
    eot_ref[...] = jnp.sum(ts, axis=1, keepdims=True).astype(jnp.int32)


def _router(xf, gamma, beta, Wg):
    outs = (
        jax.ShapeDtypeStruct((T, D), F32),
        jax.ShapeDtypeStruct((T, E), F32),
        jax.ShapeDtypeStruct((T, TOPK), F32),
        jax.ShapeDtypeStruct((T, TOPK), jnp.int32),
        jax.ShapeDtypeStruct((T, 1), jnp.int32),
        jax.ShapeDtypeStruct((T, 1), jnp.int32),
        jax.ShapeDtypeStruct((T, 1), F32),
        jax.ShapeDtypeStruct((T, 1), F32),
        jax.ShapeDtypeStruct((TILES, 1), jnp.int32),
        jax.ShapeDtypeStruct((1, 1), F32),
        jax.ShapeDtypeStruct((1, 1), F32),
        jax.ShapeDtypeStruct((1, 1), jnp.int32),
        jax.ShapeDtypeStruct((1, E), F32),
    )
    return pl.pallas_call(_router_body, out_shape=outs)(
        xf, gamma.reshape(1, D), beta.reshape(1, D), Wg)


_NC, _NS = 2, 16
_NW = _NC * _NS
_TPW = T // _NW


def _sc_mesh():
    return plsc.VectorSubcoreMesh(core_axis_name="c", subcore_axis_name="s")


def _sc_dispatch_body(xn_hbm, p0_hbm, p1_hbm, w0_hbm, w1_hbm,
                      rows_hbm, ws_hbm, idx0_v, idx1_v, rows_v,
                      wv0_v, wv1_v, sem):
    wid = jax.lax.axis_index("s") * _NC + jax.lax.axis_index("c")
    base = wid * _TPW
    pltpu.sync_copy(p0_hbm.at[pl.ds(base, _TPW)], idx0_v)
    pltpu.sync_copy(p1_hbm.at[pl.ds(base, _TPW)], idx1_v)
    pltpu.sync_copy(w0_hbm.at[pl.ds(base, _TPW)], wv0_v)
    pltpu.sync_copy(w1_hbm.at[pl.ds(base, _TPW)], wv1_v)
    pltpu.sync_copy(xn_hbm.at[pl.ds(base, _TPW)], rows_v)
    c0 = pltpu.async_copy(rows_v, rows_hbm.at[idx0_v], sem)
    c1 = pltpu.async_copy(rows_v, rows_hbm.at[idx1_v], sem)
    c2 = pltpu.async_copy(wv0_v, ws_hbm.at[idx0_v], sem)
    c3 = pltpu.async_copy(wv1_v, ws_hbm.at[idx1_v], sem)
    c0.wait()
    c1.wait()
    c2.wait()
    c3.wait()


def _dispatch(xn, p0, p1, w0, w1):
    return pl.kernel(
        _sc_dispatch_body,
        out_type=(jax.ShapeDtypeStruct((PMAX, D), F32),
                  jax.ShapeDtypeStruct((PMAX,), F32)),
        mesh=_sc_mesh(),
        scratch_types=[
            pltpu.VMEM((_TPW,), jnp.int32),
            pltpu.VMEM((_TPW,), jnp.int32),
            pltpu.VMEM((_TPW, D), F32),
            pltpu.VMEM((_TPW,), F32),
            pltpu.VMEM((_TPW,), F32),
            pltpu.SemaphoreType.DMA,
        ],
    )(xn, p0, p1, w0, w1)


_SQRT_HALF = 0.7071067811865476


def _ffn_body(eot_ref, rows_ref, ws_ref, w1_ref, b1_ref, w2_ref, b2_ref,
              h_ref):
    i = pl.program_id(0)

    @pl.when(eot_ref[i] < E)
    def _():
        a = (jnp.dot(rows_ref[...], w1_ref[0], preferred_element_type=F32)
             + b1_ref[0])
        g = a * 0.5 * (1.0 + jax.lax.erf(a * _SQRT_HALF))
        h = jnp.dot(g, w2_ref[0], preferred_element_type=F32) + b2_ref[0]
        h_ref[...] = h * ws_ref[...]


def _ffn(eot, rows, ws, w1, b1, w2, b2):
    grid_spec = pltpu.PrefetchScalarGridSpec(
        num_scalar_prefetch=1,
        grid=(TILES,),
        in_specs=[
            pl.BlockSpec((BLK, D), lambda i, eot: (i, 0)),
            pl.BlockSpec((BLK, 1), lambda i, eot: (i, 0)),
            pl.BlockSpec((1, D, FFN), lambda i, eot: (jnp.minimum(eot[i], E - 1), 0, 0)),
            pl.BlockSpec((1, 1, FFN), lambda i, eot: (jnp.minimum(eot[i], E - 1), 0, 0)),
            pl.BlockSpec((1, FFN, D), lambda i, eot: (jnp.minimum(eot[i], E - 1), 0, 0)),
            pl.BlockSpec((1, 1, D), lambda i, eot: (jnp.minimum(eot[i], E - 1), 0, 0)),
        ],
        out_specs=pl.BlockSpec((BLK, D), lambda i, eot: (i, 0)),
    )
    return pl.pallas_call(
        _ffn_body,
        grid_spec=grid_spec,
        out_shape=jax.ShapeDtypeStruct((PMAX, D), F32),
    )(eot, rows, ws.reshape(PMAX, 1), w1, b1.reshape(E, 1, FFN), w2,
      b2.reshape(E, 1, D))


def _sc_gather_body(h_hbm, p0_hbm, p1_hbm, h0_hbm, h1_hbm,
                    idx0_v, idx1_v, r0_v, r1_v, sem):
    wid = jax.lax.axis_index("s") * _NC + jax.lax.axis_index("c")
    base = wid * _TPW
    pltpu.sync_copy(p0_hbm.at[pl.ds(base, _TPW)], idx0_v)
    pltpu.sync_copy(p1_hbm.at[pl.ds(base, _TPW)], idx1_v)
    g0 = pltpu.async_copy(h_hbm.at[idx0_v], r0_v, sem)
    g1 = pltpu.async_copy(h_hbm.at[idx1_v], r1_v, sem)
    g0.wait()
    g1.wait()
    s0 = pltpu.async_copy(r0_v, h0_hbm.at[pl.ds(base, _TPW)], sem)
    s1 = pltpu.async_copy(r1_v, h1_hbm.at[pl.ds(base, _TPW)], sem)
    s0.wait()
    s1.wait()


def _gather(h, p0, p1):
    return pl.kernel(
        _sc_gather_body,
        out_type=(jax.ShapeDtypeStruct((T, D), F32),
                  jax.ShapeDtypeStruct((T, D), F32)),
        mesh=_sc_mesh(),
        scratch_types=[
            pltpu.VMEM((_TPW,), jnp.int32),
            pltpu.VMEM((_TPW,), jnp.int32),
            pltpu.VMEM((_TPW, D), F32),
            pltpu.VMEM((_TPW, D), F32),
            pltpu.SemaphoreType.DMA,
        ],
    )(h, p0, p1)


def _combine_body(x_ref, h0_ref, h1_ref, out_ref):
    out_ref[...] = x_ref[...] + h0_ref[...] + h1_ref[...]


def _combine(xf, h0, h1):
    nt = T // BLK
    return pl.pallas_call(
        _combine_body,
        grid=(nt,),
        in_specs=[
            pl.BlockSpec((BLK, D), lambda i: (i, 0)),
            pl.BlockSpec((BLK, D), lambda i: (i, 0)),
            pl.BlockSpec((BLK, D), lambda i: (i, 0)),
        ],
        out_specs=pl.BlockSpec((BLK, D), lambda i: (i, 0)),
        out_shape=jax.ShapeDtypeStruct((T, D), F32),
    )(xf, h0, h1)


def kernel(x, gamma, beta, Wg, w1, b1, w2, b2):
    xf = x.reshape(T, D)
    (xn, logits, dw, ei, p0, p1, dw0, dw1, eot,
     lbl, rel, nd, usage) = _router(xf, gamma, beta, Wg)
    p0f = p0.reshape(T)
    p1f = p1.reshape(T)
    rows, ws = _dispatch(xn, p0f, p1f, dw0.reshape(T), dw1.reshape(T))
    h = _ffn(eot.reshape(TILES), rows, ws, w1, b1, w2, b2)
    h0, h1 = _gather(h, p0f, p1f)
    out = _combine(xf, h0, h1).reshape(B, S, D)
    return (out, dw, ei, logits, lbl[0, 0], rel[0, 0], nd[0, 0], usage[0])


# speedup vs baseline: 1.2514x; 1.1907x over previous
import functools

import jax
import jax.numpy as jnp
from jax.experimental import pallas as pl
from jax.experimental.pallas import tpu as pltpu
from jax.experimental.pallas import tpu_sc as plsc

B, S, D = 1, 2048, 768
E, TOPK, FFN = 8, 2, 3072
T = B * S
BLK = 256
TILES = (T * TOPK + E * BLK) // BLK
PMAX = TILES * BLK
CAPACITY = max(1, int(T * TOPK * 1.25) // E)
F32 = jnp.float32


def _router_body(x_ref, g_ref, b_ref, wg_ref,
                 xn_ref, logits_ref, dw_ref, ei_ref, p0_ref, p1_ref,
                 w0_ref, w1_ref, eot_ref,
                 lbl_ref, rel_ref, nd_ref, usage_ref):
    x = x_ref[...]
    mean = jnp.mean(x, axis=1, keepdims=True)
    cx = x - mean
    var = jnp.mean(cx * cx, axis=1, keepdims=True)
    xn = cx * jax.lax.rsqrt(var + 1e-5) * g_ref[...] + b_ref[...]
    xn_ref[...] = xn

    logits = jnp.dot(xn, wg_ref[...], preferred_element_type=F32)
    logits_ref[...] = logits
    m = jnp.max(logits, axis=1, keepdims=True)
    ex = jnp.exp(logits - m)
    den = jnp.sum(ex, axis=1, keepdims=True)
    probs = ex / den

    idx = jax.lax.broadcasted_iota(jnp.int32, (T, E), 1)
    p0 = jnp.max(probs, axis=1, keepdims=True)
    is0 = probs == p0
    e0 = jnp.min(jnp.where(is0, idx, E), axis=1, keepdims=True)
    pm = jnp.where(idx == e0, -1.0, probs)
    p1 = jnp.max(pm, axis=1, keepdims=True)
    is1 = pm == p1
    e1 = jnp.min(jnp.where(is1, idx, E), axis=1, keepdims=True)
    dw_ref[...] = jnp.concatenate([p0, p1], axis=1)
    ei_ref[...] = jnp.concatenate([e0, e1], axis=1)

    logp = logits - m - jnp.log(den)
    ent = -jnp.sum(probs * logp, axis=1, keepdims=True)
    entropy = jnp.mean(ent)
    rel_ref[...] = jnp.maximum(jnp.log(jnp.float32(E)) - entropy,
                               0.0).reshape(1, 1)
    avg_probs = jnp.mean(probs, axis=0, keepdims=True)

    oh0 = (idx == e0).astype(F32)
    oh1 = (idx == e1).astype(F32)
    r = jax.lax.broadcasted_iota(jnp.int32, (T, T), 0)
    c = jax.lax.broadcasted_iota(jnp.int32, (T, T), 1)
    tri = (r >= c).astype(F32)
    cums = jnp.dot(tri, jnp.concatenate([oh0, oh1], axis=1),
                   preferred_element_type=F32)
    cum0 = cums[:, :E]
    cum1 = cums[:, E:]
    counts0 = cum0[T - 1:T, :]
    counts1 = cum1[T - 1:T, :]
    counts = counts0 + counts1

    nd_ref[...] = jnp.sum((counts > CAPACITY).astype(F32)
                          ).astype(jnp.int32).reshape(1, 1)
    usage = counts / T
    usage_ref[...] = usage
    lbl_ref[...] = (jnp.sum(usage * avg_probs) * E).reshape(1, 1)

    aligned = jnp.ceil(counts / BLK) * BLK
    r8 = jax.lax.broadcasted_iota(jnp.int32, (E, E), 0)
    c8 = jax.lax.broadcasted_iota(jnp.int32, (E, E), 1)
    up = (r8 < c8).astype(F32)
    offs = jnp.dot(aligned, up, preferred_element_type=F32)
    ends = offs + aligned

    pos0 = jnp.sum(oh0 * (cum0 + offs), axis=1, keepdims=True) - 1.0
    pos1 = jnp.sum(oh1 * (cum1 + counts0 + offs), axis=1, keepdims=True) - 1.0
    p0_ref[...] = pos0.astype(jnp.int32)
    p1_ref[...] = pos1.astype(jnp.int32)
    w0_ref[...] = p0
    w1_ref[...] = p1

    ts = (jax.lax.broadcasted_iota(jnp.int32, (TILES, E), 0).astype(F32) * BLK
          >= jax.lax.broadcast_in_dim(ends, (TILES, E), (0, 1))).astype(F32)
    eot_ref[...] = jnp.sum(ts, axis=1, keepdims=True).astype(jnp.int32)


def _router(xf, gamma, beta, Wg):
    outs = (
        jax.ShapeDtypeStruct((T, D), F32),
        jax.ShapeDtypeStruct((T, E), F32),
        jax.ShapeDtypeStruct((T, TOPK), F32),
        jax.ShapeDtypeStruct((T, TOPK), jnp.int32),
        jax.ShapeDtypeStruct((T, 1), jnp.int32),
        jax.ShapeDtypeStruct((T, 1), jnp.int32),
        jax.ShapeDtypeStruct((T, 1), F32),
        jax.ShapeDtypeStruct((T, 1), F32),
        jax.ShapeDtypeStruct((TILES, 1), jnp.int32),
        jax.ShapeDtypeStruct((1, 1), F32),
        jax.ShapeDtypeStruct((1, 1), F32),
        jax.ShapeDtypeStruct((1, 1), jnp.int32),
        jax.ShapeDtypeStruct((1, E), F32),
    )
    return pl.pallas_call(_router_body, out_shape=outs)(
        xf, gamma.reshape(1, D), beta.reshape(1, D), Wg)


_NC, _NS = 2, 16
_NW = _NC * _NS
_TPW = T // _NW


def _sc_mesh():
    return plsc.VectorSubcoreMesh(core_axis_name="c", subcore_axis_name="s")


def _sc_dispatch_body(xn_hbm, p0_hbm, p1_hbm, rows_hbm, idx0_v, idx1_v,
                      rows_v, sem):
    wid = jax.lax.axis_index("s") * _NC + jax.lax.axis_index("c")
    base = wid * _TPW
    pltpu.sync_copy(p0_hbm.at[pl.ds(base, _TPW)], idx0_v)
    pltpu.sync_copy(p1_hbm.at[pl.ds(base, _TPW)], idx1_v)
    pltpu.sync_copy(xn_hbm.at[pl.ds(base, _TPW)], rows_v)
    c0 = pltpu.async_copy(rows_v, rows_hbm.at[idx0_v], sem)
    c1 = pltpu.async_copy(rows_v, rows_hbm.at[idx1_v], sem)
    c0.wait()
    c1.wait()


def _dispatch(xn, p0, p1):
    return pl.kernel(
        _sc_dispatch_body,
        out_type=jax.ShapeDtypeStruct((PMAX, D), F32),
        mesh=_sc_mesh(),
        scratch_types=[
            pltpu.VMEM((_TPW,), jnp.int32),
            pltpu.VMEM((_TPW,), jnp.int32),
            pltpu.VMEM((_TPW, D), F32),
            pltpu.SemaphoreType.DMA,
        ],
    )(xn, p0, p1)


_SQRT_HALF = 0.7071067811865476


def _ffn_body(eot_ref, rows_ref, w1_ref, b1_ref, w2_ref, b2_ref, h_ref):
    i = pl.program_id(0)

    @pl.when(eot_ref[i] < E)
    def _():
        a = (jnp.dot(rows_ref[...], w1_ref[0], preferred_element_type=F32)
             + b1_ref[0])
        g = a * 0.5 * (1.0 + jax.lax.erf(a * _SQRT_HALF))
        h_ref[...] = jnp.dot(g, w2_ref[0], preferred_element_type=F32) + b2_ref[0]


def _ffn(eot, rows, w1, b1, w2, b2):
    grid_spec = pltpu.PrefetchScalarGridSpec(
        num_scalar_prefetch=1,
        grid=(TILES,),
        in_specs=[
            pl.BlockSpec((BLK, D), lambda i, eot: (i, 0)),
            pl.BlockSpec((1, D, FFN), lambda i, eot: (jnp.minimum(eot[i], E - 1), 0, 0)),
            pl.BlockSpec((1, 1, FFN), lambda i, eot: (jnp.minimum(eot[i], E - 1), 0, 0)),
            pl.BlockSpec((1, FFN, D), lambda i, eot: (jnp.minimum(eot[i], E - 1), 0, 0)),
            pl.BlockSpec((1, 1, D), lambda i, eot: (jnp.minimum(eot[i], E - 1), 0, 0)),
        ],
        out_specs=pl.BlockSpec((BLK, D), lambda i, eot: (i, 0)),
    )
    return pl.pallas_call(
        _ffn_body,
        grid_spec=grid_spec,
        out_shape=jax.ShapeDtypeStruct((PMAX, D), F32),
    )(eot, rows, w1, b1.reshape(E, 1, FFN), w2, b2.reshape(E, 1, D))


def _sc_gather_body(h_hbm, p0_hbm, p1_hbm, h0_hbm, h1_hbm,
                    idx0_v, idx1_v, r0_v, r1_v, sem):
    wid = jax.lax.axis_index("s") * _NC + jax.lax.axis_index("c")
    base = wid * _TPW
    pltpu.sync_copy(p0_hbm.at[pl.ds(base, _TPW)], idx0_v)
    pltpu.sync_copy(p1_hbm.at[pl.ds(base, _TPW)], idx1_v)
    g0 = pltpu.async_copy(h_hbm.at[idx0_v], r0_v, sem)
    g1 = pltpu.async_copy(h_hbm.at[idx1_v], r1_v, sem)
    g0.wait()
    g1.wait()
    s0 = pltpu.async_copy(r0_v, h0_hbm.at[pl.ds(base, _TPW)], sem)
    s1 = pltpu.async_copy(r1_v, h1_hbm.at[pl.ds(base, _TPW)], sem)
    s0.wait()
    s1.wait()


def _gather(h, p0, p1):
    return pl.kernel(
        _sc_gather_body,
        out_type=(jax.ShapeDtypeStruct((T, D), F32),
                  jax.ShapeDtypeStruct((T, D), F32)),
        mesh=_sc_mesh(),
        scratch_types=[
            pltpu.VMEM((_TPW,), jnp.int32),
            pltpu.VMEM((_TPW,), jnp.int32),
            pltpu.VMEM((_TPW, D), F32),
            pltpu.VMEM((_TPW, D), F32),
            pltpu.SemaphoreType.DMA,
        ],
    )(h, p0, p1)


def _combine_body(dw_ref, x_ref, h0_ref, h1_ref, out_ref):
    out_ref[...] = (x_ref[...] + dw_ref[:, 0:1] * h0_ref[...]
                    + dw_ref[:, 1:2] * h1_ref[...])


def _combine(dw, xf, h0, h1):
    nt = T // BLK
    return pl.pallas_call(
        _combine_body,
        grid=(nt,),
        in_specs=[
            pl.BlockSpec((BLK, TOPK), lambda i: (i, 0)),
            pl.BlockSpec((BLK, D), lambda i: (i, 0)),
            pl.BlockSpec((BLK, D), lambda i: (i, 0)),
            pl.BlockSpec((BLK, D), lambda i: (i, 0)),
        ],
        out_specs=pl.BlockSpec((BLK, D), lambda i: (i, 0)),
        out_shape=jax.ShapeDtypeStruct((T, D), F32),
    )(dw, xf, h0, h1)


def kernel(x, gamma, beta, Wg, w1, b1, w2, b2):
    xf = x.reshape(T, D)
    (xn, logits, dw, ei, p0, p1, dw0, dw1, eot,
     lbl, rel, nd, usage) = _router(xf, gamma, beta, Wg)
    p0f = p0.reshape(T)
    p1f = p1.reshape(T)
    rows = _dispatch(xn, p0f, p1f)
    h = _ffn(eot.reshape(TILES), rows, w1, b1, w2, b2)
    h0, h1 = _gather(h, p0f, p1f)
    out = _combine(dw, xf, h0, h1).reshape(B, S, D)
    return (out, dw, ei, logits, lbl[0, 0], rel[0, 0], nd[0, 0], usage[0])


# speedup vs baseline: 1.2743x; 1.0183x over previous
import functools

import jax
import jax.numpy as jnp
from jax.experimental import pallas as pl
from jax.experimental.pallas import tpu as pltpu
from jax.experimental.pallas import tpu_sc as plsc

B, S, D = 1, 2048, 768
E, TOPK, FFN = 8, 2, 3072
T = B * S
BLK = 256
TILES = (T * TOPK + E * BLK) // BLK
PMAX = TILES * BLK
CAPACITY = max(1, int(T * TOPK * 1.25) // E)
F32 = jnp.float32


def _router_body(x_ref, g_ref, b_ref, wg_ref,
                 xn_ref, logits_ref, dw_ref, ei_ref, p0_ref, p1_ref,
                 w0_ref, w1_ref, eot_ref,
                 lbl_ref, rel_ref, nd_ref, usage_ref):
    x = x_ref[...]
    mean = jnp.mean(x, axis=1, keepdims=True)
    cx = x - mean
    var = jnp.mean(cx * cx, axis=1, keepdims=True)
    xn = cx * jax.lax.rsqrt(var + 1e-5) * g_ref[...] + b_ref[...]
    xn_ref[...] = _pack_halves(xn)

    logits = jnp.dot(xn, wg_ref[...], preferred_element_type=F32)
    logits_ref[...] = logits
    m = jnp.max(logits, axis=1, keepdims=True)
    ex = jnp.exp(logits - m)
    den = jnp.sum(ex, axis=1, keepdims=True)
    probs = ex / den

    idx = jax.lax.broadcasted_iota(jnp.int32, (T, E), 1)
    p0 = jnp.max(probs, axis=1, keepdims=True)
    is0 = probs == p0
    e0 = jnp.min(jnp.where(is0, idx, E), axis=1, keepdims=True)
    pm = jnp.where(idx == e0, -1.0, probs)
    p1 = jnp.max(pm, axis=1, keepdims=True)
    is1 = pm == p1
    e1 = jnp.min(jnp.where(is1, idx, E), axis=1, keepdims=True)
    dw_ref[...] = jnp.concatenate([p0, p1], axis=1)
    ei_ref[...] = jnp.concatenate([e0, e1], axis=1)

    logp = logits - m - jnp.log(den)
    ent = -jnp.sum(probs * logp, axis=1, keepdims=True)
    entropy = jnp.mean(ent)
    rel_ref[...] = jnp.maximum(jnp.log(jnp.float32(E)) - entropy,
                               0.0).reshape(1, 1)
    avg_probs = jnp.mean(probs, axis=0, keepdims=True)

    oh0 = (idx == e0).astype(F32)
    oh1 = (idx == e1).astype(F32)
    r = jax.lax.broadcasted_iota(jnp.int32, (T, T), 0)
    c = jax.lax.broadcasted_iota(jnp.int32, (T, T), 1)
    tri = (r >= c).astype(F32)
    cums = jnp.dot(tri, jnp.concatenate([oh0, oh1], axis=1),
                   preferred_element_type=F32)
    cum0 = cums[:, :E]
    cum1 = cums[:, E:]
    counts0 = cum0[T - 1:T, :]
    counts1 = cum1[T - 1:T, :]
    counts = counts0 + counts1

    nd_ref[...] = jnp.sum((counts > CAPACITY).astype(F32)
                          ).astype(jnp.int32).reshape(1, 1)
    usage = counts / T
    usage_ref[...] = usage
    lbl_ref[...] = (jnp.sum(usage * avg_probs) * E).reshape(1, 1)

    aligned = jnp.ceil(counts / BLK) * BLK
    r8 = jax.lax.broadcasted_iota(jnp.int32, (E, E), 0)
    c8 = jax.lax.broadcasted_iota(jnp.int32, (E, E), 1)
    up = (r8 < c8).astype(F32)
    offs = jnp.dot(aligned, up, preferred_element_type=F32)
    ends = offs + aligned

    pos0 = jnp.sum(oh0 * (cum0 + offs), axis=1, keepdims=True) - 1.0
    pos1 = jnp.sum(oh1 * (cum1 + counts0 + offs), axis=1, keepdims=True) - 1.0
    p0_ref[...] = pos0.astype(jnp.int32)
    p1_ref[...] = pos1.astype(jnp.int32)
    w0_ref[...] = p0
    w1_ref[...] = p1

    ts = (jax.lax.broadcasted_iota(jnp.int32, (TILES, E), 0).astype(F32) * BLK
          >= jax.lax.broadcast_in_dim(ends, (TILES, E), (0, 1))).astype(F32)
    eot_ref[...] = jnp.sum(ts, axis=1, keepdims=True).astype(jnp.int32)


def _router(xf, gamma, beta, Wg):
    outs = (
        jax.ShapeDtypeStruct((T, _HD), jnp.int32),
        jax.ShapeDtypeStruct((T, E), F32),
        jax.ShapeDtypeStruct((T, TOPK), F32),
        jax.ShapeDtypeStruct((T, TOPK), jnp.int32),
        jax.ShapeDtypeStruct((T, 1), jnp.int32),
        jax.ShapeDtypeStruct((T, 1), jnp.int32),
        jax.ShapeDtypeStruct((T, 1), F32),
        jax.ShapeDtypeStruct((T, 1), F32),
        jax.ShapeDtypeStruct((TILES, 1), jnp.int32),
        jax.ShapeDtypeStruct((1, 1), F32),
        jax.ShapeDtypeStruct((1, 1), F32),
        jax.ShapeDtypeStruct((1, 1), jnp.int32),
        jax.ShapeDtypeStruct((1, E), F32),
    )
    return pl.pallas_call(_router_body, out_shape=outs)(
        xf, gamma.reshape(1, D), beta.reshape(1, D), Wg)


_NC, _NS = 2, 16
_NW = _NC * _NS
_TPW = T // _NW


def _sc_mesh():
    return plsc.VectorSubcoreMesh(core_axis_name="c", subcore_axis_name="s")


def _sc_dispatch_body(xn_hbm, p0_hbm, p1_hbm, rows_hbm, idx0_v, idx1_v,
                      rows_v, sem):
    wid = jax.lax.axis_index("s") * _NC + jax.lax.axis_index("c")
    base = wid * _TPW
    pltpu.sync_copy(p0_hbm.at[pl.ds(base, _TPW)], idx0_v)
    pltpu.sync_copy(p1_hbm.at[pl.ds(base, _TPW)], idx1_v)
    pltpu.sync_copy(xn_hbm.at[pl.ds(base, _TPW)], rows_v)
    c0 = pltpu.async_copy(rows_v, rows_hbm.at[idx0_v], sem)
    c1 = pltpu.async_copy(rows_v, rows_hbm.at[idx1_v], sem)
    c0.wait()
    c1.wait()


def _dispatch(xn, p0, p1):
    return pl.kernel(
        _sc_dispatch_body,
        out_type=jax.ShapeDtypeStruct((PMAX, D // 2), jnp.int32),
        mesh=_sc_mesh(),
        scratch_types=[
            pltpu.VMEM((_TPW,), jnp.int32),
            pltpu.VMEM((_TPW,), jnp.int32),
            pltpu.VMEM((_TPW, D // 2), jnp.int32),
            pltpu.SemaphoreType.DMA,
        ],
    )(xn, p0, p1)


_SQRT_HALF = 0.7071067811865476
_HD = D // 2


def _to_bf16_bits(f):
    b = jax.lax.bitcast_convert_type(f, jnp.int32)
    b = b + 0x7FFF + jax.lax.bitwise_and(
        jax.lax.shift_right_logical(b, 16), jnp.int32(1))
    return jax.lax.bitwise_and(b, jnp.int32(-65536))


def _pack_halves(f):
    lo = _to_bf16_bits(f[:, :_HD])
    hi = _to_bf16_bits(f[:, _HD:])
    return jax.lax.bitwise_or(jax.lax.shift_right_logical(lo, 16), hi)


def _unpack_halves(v):
    lo = jax.lax.bitcast_convert_type(jax.lax.shift_left(v, 16), F32)
    hi = jax.lax.bitcast_convert_type(
        jax.lax.bitwise_and(v, jnp.int32(-65536)), F32)
    return lo, hi


def _ffn_body(eot_ref, rows_ref, w1_ref, b1_ref, w2_ref, b2_ref, h_ref):
    i = pl.program_id(0)

    @pl.when(eot_ref[i] < E)
    def _():
        lo, hi = _unpack_halves(rows_ref[...])
        a = (jnp.dot(lo, w1_ref[0, :_HD], preferred_element_type=F32)
             + jnp.dot(hi, w1_ref[0, _HD:], preferred_element_type=F32)
             + b1_ref[0])
        g = a * 0.5 * (1.0 + jax.lax.erf(a * _SQRT_HALF))
        h = jnp.dot(g, w2_ref[0], preferred_element_type=F32) + b2_ref[0]
        h_ref[...] = _pack_halves(h)


def _ffn(eot, rows, w1, b1, w2, b2):
    grid_spec = pltpu.PrefetchScalarGridSpec(
        num_scalar_prefetch=1,
        grid=(TILES,),
        in_specs=[
            pl.BlockSpec((BLK, _HD), lambda i, eot: (i, 0)),
            pl.BlockSpec((1, D, FFN), lambda i, eot: (jnp.minimum(eot[i], E - 1), 0, 0)),
            pl.BlockSpec((1, 1, FFN), lambda i, eot: (jnp.minimum(eot[i], E - 1), 0, 0)),
            pl.BlockSpec((1, FFN, D), lambda i, eot: (jnp.minimum(eot[i], E - 1), 0, 0)),
            pl.BlockSpec((1, 1, D), lambda i, eot: (jnp.minimum(eot[i], E - 1), 0, 0)),
        ],
        out_specs=pl.BlockSpec((BLK, _HD), lambda i, eot: (i, 0)),
    )
    return pl.pallas_call(
        _ffn_body,
        grid_spec=grid_spec,
        out_shape=jax.ShapeDtypeStruct((PMAX, _HD), jnp.int32),
    )(eot, rows, w1, b1.reshape(E, 1, FFN), w2, b2.reshape(E, 1, D))


def _sc_gather_body(h_hbm, p0_hbm, p1_hbm, h0_hbm, h1_hbm,
                    idx0_v, idx1_v, r0_v, r1_v, sem):
    wid = jax.lax.axis_index("s") * _NC + jax.lax.axis_index("c")
    base = wid * _TPW
    pltpu.sync_copy(p0_hbm.at[pl.ds(base, _TPW)], idx0_v)
    pltpu.sync_copy(p1_hbm.at[pl.ds(base, _TPW)], idx1_v)
    g0 = pltpu.async_copy(h_hbm.at[idx0_v], r0_v, sem)
    g1 = pltpu.async_copy(h_hbm.at[idx1_v], r1_v, sem)
    g0.wait()
    g1.wait()
    s0 = pltpu.async_copy(r0_v, h0_hbm.at[pl.ds(base, _TPW)], sem)
    s1 = pltpu.async_copy(r1_v, h1_hbm.at[pl.ds(base, _TPW)], sem)
    s0.wait()
    s1.wait()


def _gather(h, p0, p1):
    return pl.kernel(
        _sc_gather_body,
        out_type=(jax.ShapeDtypeStruct((T, D // 2), jnp.int32),
                  jax.ShapeDtypeStruct((T, D // 2), jnp.int32)),
        mesh=_sc_mesh(),
        scratch_types=[
            pltpu.VMEM((_TPW,), jnp.int32),
            pltpu.VMEM((_TPW,), jnp.int32),
            pltpu.VMEM((_TPW, D // 2), jnp.int32),
            pltpu.VMEM((_TPW, D // 2), jnp.int32),
            pltpu.SemaphoreType.DMA,
        ],
    )(h, p0, p1)


def _combine_body(dw_ref, x_ref, h0_ref, h1_ref, out_ref):
    w0 = dw_ref[:, 0:1]
    w1c = dw_ref[:, 1:2]
    h0lo, h0hi = _unpack_halves(h0_ref[...])
    h1lo, h1hi = _unpack_halves(h1_ref[...])
    out_ref[:, :_HD] = x_ref[:, :_HD] + w0 * h0lo + w1c * h1lo
    out_ref[:, _HD:] = x_ref[:, _HD:] + w0 * h0hi + w1c * h1hi


def _combine(dw, xf, h0, h1):
    nt = T // BLK
    return pl.pallas_call(
        _combine_body,
        grid=(nt,),
        in_specs=[
            pl.BlockSpec((BLK, TOPK), lambda i: (i, 0)),
            pl.BlockSpec((BLK, D), lambda i: (i, 0)),
            pl.BlockSpec((BLK, _HD), lambda i: (i, 0)),
            pl.BlockSpec((BLK, _HD), lambda i: (i, 0)),
        ],
        out_specs=pl.BlockSpec((BLK, D), lambda i: (i, 0)),
        out_shape=jax.ShapeDtypeStruct((T, D), F32),
    )(dw, xf, h0, h1)


def kernel(x, gamma, beta, Wg, w1, b1, w2, b2):
    xf = x.reshape(T, D)
    (xn, logits, dw, ei, p0, p1, dw0, dw1, eot,
     lbl, rel, nd, usage) = _router(xf, gamma, beta, Wg)
    p0f = p0.reshape(T)
    p1f = p1.reshape(T)
    rows = _dispatch(xn, p0f, p1f)
    h = _ffn(eot.reshape(TILES), rows, w1, b1, w2, b2)
    h0, h1 = _gather(h, p0f, p1f)
    out = _combine(dw, xf, h0, h1).reshape(B, S, D)
    return (out, dw, ei, logits, lbl[0, 0], rel[0, 0], nd[0, 0], usage[0])
